# baseline (speedup 1.0000x reference)
import functools

import jax
import jax.numpy as jnp
from jax.experimental import pallas as pl
from jax.experimental.pallas import tpu as pltpu

_BM = 4096


def _fused_heads_kernel(x_ref, wc_ref, wb_ref, bc_ref, bb_ref,
                        st_ref, dt_ref):
    x = x_ref[...].astype(jnp.bfloat16)
    dn = (((1,), (1,)), ((), ()))
    st_ref[...] = jax.lax.dot_general(
        wc_ref[...], x, dn, preferred_element_type=jnp.float32) + bc_ref[...]
    dt_ref[...] = jax.lax.dot_general(
        wb_ref[...], x, dn, preferred_element_type=jnp.float32) + bb_ref[...]


@functools.partial(jax.jit, static_argnames=("interpret",))
def _run(x, W_cls, b_cls, W_box, b_box, interpret=False):
    n, d = x.shape
    c1 = W_cls.shape[0]
    c4 = W_box.shape[0]
    grid = (pl.cdiv(n, _BM),)
    st, dt = pl.pallas_call(
        _fused_heads_kernel,
        grid=grid,
        in_specs=[
            pl.BlockSpec((_BM, d), lambda i: (i, 0)),
            pl.BlockSpec((c1, d), lambda i: (0, 0)),
            pl.BlockSpec((c4, d), lambda i: (0, 0)),
            pl.BlockSpec((c1, 1), lambda i: (0, 0)),
            pl.BlockSpec((c4, 1), lambda i: (0, 0)),
        ],
        out_specs=[
            pl.BlockSpec((c1, _BM), lambda i: (0, i)),
            pl.BlockSpec((c4, _BM), lambda i: (0, i)),
        ],
        out_shape=[
            jax.ShapeDtypeStruct((c1, n), jnp.float32),
            jax.ShapeDtypeStruct((c4, n), jnp.float32),
        ],
        compiler_params=pltpu.CompilerParams(
            dimension_semantics=("parallel",),
            vmem_limit_bytes=100 * 1024 * 1024),
        interpret=interpret,
    )(x, W_cls.astype(jnp.bfloat16), W_box.astype(jnp.bfloat16),
      b_cls.reshape(c1, 1), b_box.reshape(c4, 1))
    return st.T, dt.T


def kernel(x, W_cls, b_cls, W_box, b_box):
    if x.ndim > 2:
        x = x.reshape(x.shape[0], -1)
    return _run(x, W_cls, b_cls, W_box, b_box)


# speedup vs baseline: 1.0915x; 1.0915x over previous
import functools

import jax
import jax.numpy as jnp
from jax.experimental import pallas as pl
from jax.experimental.pallas import tpu as pltpu

_BM = 2048


def _fused_heads_kernel(x_ref, wc_ref, wb_ref, bc_ref, bb_ref,
                        st_ref, dt_ref):
    x = x_ref[...]
    dn = (((1,), (1,)), ((), ()))
    st_ref[...] = jax.lax.dot_general(
        wc_ref[...], x, dn, preferred_element_type=jnp.float32) + bc_ref[...]
    dt_ref[...] = jax.lax.dot_general(
        wb_ref[...], x, dn, preferred_element_type=jnp.float32) + bb_ref[...]


@functools.partial(jax.jit, static_argnames=("interpret",))
def _run(x, W_cls, b_cls, W_box, b_box, interpret=False):
    n, d = x.shape
    c1 = W_cls.shape[0]
    c4 = W_box.shape[0]
    grid = (pl.cdiv(n, _BM),)
    st, dt = pl.pallas_call(
        _fused_heads_kernel,
        grid=grid,
        in_specs=[
            pl.BlockSpec((_BM, d), lambda i: (i, 0)),
            pl.BlockSpec((c1, d), lambda i: (0, 0)),
            pl.BlockSpec((c4, d), lambda i: (0, 0)),
            pl.BlockSpec((c1, 1), lambda i: (0, 0)),
            pl.BlockSpec((c4, 1), lambda i: (0, 0)),
        ],
        out_specs=[
            pl.BlockSpec((c1, _BM), lambda i: (0, i)),
            pl.BlockSpec((c4, _BM), lambda i: (0, i)),
        ],
        out_shape=[
            jax.ShapeDtypeStruct((c1, n), jnp.float32),
            jax.ShapeDtypeStruct((c4, n), jnp.float32),
        ],
        compiler_params=pltpu.CompilerParams(
            dimension_semantics=("parallel",),
            vmem_limit_bytes=100 * 1024 * 1024),
        interpret=interpret,
    )(x, W_cls, W_box, b_cls.reshape(c1, 1), b_box.reshape(c4, 1))
    return st.T, dt.T


def kernel(x, W_cls, b_cls, W_box, b_box):
    if x.ndim > 2:
        x = x.reshape(x.shape[0], -1)
    return _run(x, W_cls, b_cls, W_box, b_box)


# speedup vs baseline: 1.0997x; 1.0075x over previous
import functools

import jax
import jax.numpy as jnp
from jax.experimental import pallas as pl
from jax.experimental.pallas import tpu as pltpu

_BM = 2560


def _fused_heads_kernel(x_ref, wc_ref, wb_ref, bc_ref, bb_ref,
                        st_ref, dt_ref):
    x = x_ref[...]
    dn = (((1,), (1,)), ((), ()))
    st_ref[...] = jax.lax.dot_general(
        wc_ref[...], x, dn, preferred_element_type=jnp.float32) + bc_ref[...]
    dt_ref[...] = jax.lax.dot_general(
        wb_ref[...], x, dn, preferred_element_type=jnp.float32) + bb_ref[...]


@functools.partial(jax.jit, static_argnames=("interpret",))
def _run(x, W_cls, b_cls, W_box, b_box, interpret=False):
    n, d = x.shape
    c1 = W_cls.shape[0]
    c4 = W_box.shape[0]
    grid = (pl.cdiv(n, _BM),)
    st, dt = pl.pallas_call(
        _fused_heads_kernel,
        grid=grid,
        in_specs=[
            pl.BlockSpec((_BM, d), lambda i: (i, 0)),
            pl.BlockSpec((c1, d), lambda i: (0, 0)),
            pl.BlockSpec((c4, d), lambda i: (0, 0)),
            pl.BlockSpec((c1, 1), lambda i: (0, 0)),
            pl.BlockSpec((c4, 1), lambda i: (0, 0)),
        ],
        out_specs=[
            pl.BlockSpec((c1, _BM), lambda i: (0, i)),
            pl.BlockSpec((c4, _BM), lambda i: (0, i)),
        ],
        out_shape=[
            jax.ShapeDtypeStruct((c1, n), jnp.float32),
            jax.ShapeDtypeStruct((c4, n), jnp.float32),
        ],
        compiler_params=pltpu.CompilerParams(
            dimension_semantics=("parallel",),
            vmem_limit_bytes=100 * 1024 * 1024),
        interpret=interpret,
    )(x, W_cls, W_box, b_cls.reshape(c1, 1), b_box.reshape(c4, 1))
    return st.T, dt.T


def kernel(x, W_cls, b_cls, W_box, b_box):
    if x.ndim > 2:
        x = x.reshape(x.shape[0], -1)
    return _run(x, W_cls, b_cls, W_box, b_box)


# speedup vs baseline: 1.1065x; 1.0062x over previous
import functools

import jax
import jax.numpy as jnp
from jax.experimental import pallas as pl
from jax.experimental.pallas import tpu as pltpu

_BM = 2560


def _fused_heads_kernel(x_ref, wc_ref, wb_ref, bc_ref, bb_ref,
                        st_ref, dt_ref):
    x = x_ref[...]
    dn = (((1,), (1,)), ((), ()))
    st_ref[...] = jax.lax.dot_general(
        wc_ref[...], x, dn, preferred_element_type=jnp.float32) + bc_ref[...]
    dt_ref[...] = jax.lax.dot_general(
        wb_ref[...], x, dn, preferred_element_type=jnp.float32) + bb_ref[...]


@functools.partial(jax.jit, static_argnames=("interpret",))
def _run(x, W_cls, b_cls, W_box, b_box, interpret=False):
    n, d = x.shape
    c1 = W_cls.shape[0]
    c4 = W_box.shape[0]
    grid = (pl.cdiv(n, _BM),)
    st, dt = pl.pallas_call(
        _fused_heads_kernel,
        grid=grid,
        in_specs=[
            pl.BlockSpec((_BM, d), lambda i: (i, 0)),
            pl.BlockSpec((c1, d), lambda i: (0, 0)),
            pl.BlockSpec((c4, d), lambda i: (0, 0)),
            pl.BlockSpec((c1, 1), lambda i: (0, 0)),
            pl.BlockSpec((c4, 1), lambda i: (0, 0)),
        ],
        out_specs=[
            pl.BlockSpec((c1, _BM), lambda i: (0, i)),
            pl.BlockSpec((c4, _BM), lambda i: (0, i)),
        ],
        out_shape=[
            jax.ShapeDtypeStruct((c1, n), jnp.float32),
            jax.ShapeDtypeStruct((c4, n), jnp.float32),
        ],
        compiler_params=pltpu.CompilerParams(
            dimension_semantics=("arbitrary",),
            vmem_limit_bytes=100 * 1024 * 1024),
        interpret=interpret,
    )(x, W_cls, W_box, b_cls.reshape(c1, 1), b_box.reshape(c4, 1))
    return st.T, dt.T


def kernel(x, W_cls, b_cls, W_box, b_box):
    if x.ndim > 2:
        x = x.reshape(x.shape[0], -1)
    return _run(x, W_cls, b_cls, W_box, b_box)
